# baseline (speedup 1.0000x reference)
import functools

import jax
import jax.numpy as jnp
from jax import lax
from jax.experimental import pallas as pl
from jax.experimental.pallas import tpu as pltpu
from jax.experimental.pallas import tpu_sc as plsc

N = 10000
D = 128
E = 320000
L = 3

NUM_SC = 2
NUM_TILES = 16
NW = NUM_SC * NUM_TILES

CH = 32
K = 4
R0 = 152
R1 = 8
RTOT = R0 + R1
E_PAD = NUM_TILES * RTOT * K * CH
N_PAD = 10112
RPT = N_PAD // NUM_TILES
NZF = RPT // CH
NZT = RPT - NZF * CH


def _sc_segment_sum(m, idx5):
    mesh = plsc.VectorSubcoreMesh(core_axis_name="c", subcore_axis_name="s")

    @functools.partial(
        pl.kernel,
        out_type=jax.ShapeDtypeStruct((NUM_SC, N_PAD, D), jnp.float32),
        mesh=mesh,
        scratch_types=[
            [pltpu.VMEM((2, K, CH), jnp.int32) for _ in range(4)],
            [pltpu.VMEM((CH, D), jnp.float32) for _ in range(2 * K)],
            pltpu.VMEM_SHARED((N_PAD, D), jnp.float32),
            [pltpu.SemaphoreType.DMA for _ in range(2 * K)],
            [pltpu.SemaphoreType.DMA for _ in range(2 * K)],
            [pltpu.SemaphoreType.DMA for _ in range(4)],
            pltpu.SemaphoreType.DMA,
        ],
    )
    def kernel_fn(m_hbm, idx_hbm, out_hbm, irb, rows, acc, gs, ss, isem, zsem):
        cid = lax.axis_index("c")
        mloc = m_hbm
        sid = lax.axis_index("s")
        rbase = cid * R0
        nrounds = jnp.where(cid == 0, R0, R1)

        pltpu.async_copy(idx_hbm.at[sid, rbase + 0], irb[0], isem[0])
        pltpu.async_copy(idx_hbm.at[sid, rbase + 1], irb[1], isem[1])

        zbuf = rows[K]
        @pl.loop(0, CH)
        def _(r):
            @pl.loop(0, D, step=16)
            def _(c):
                zbuf[r, pl.ds(c, 16)] = jnp.zeros((16,), jnp.float32)

        for z in range(NZF):
            pltpu.async_copy(zbuf, acc.at[pl.ds(sid * RPT + z * CH, CH)], zsem)
        pltpu.async_copy(zbuf.at[pl.ds(0, NZT)],
                         acc.at[pl.ds(sid * RPT + NZF * CH, NZT)], zsem)

        pltpu.make_async_copy(idx_hbm.at[sid, rbase + 0], irb[0], isem[0]).wait()
        for k in range(K):
            pltpu.async_copy(mloc.at[irb[0].at[0, k]], rows[k], gs[k])

        for z in range(NZF):
            pltpu.make_async_copy(
                zbuf, acc.at[pl.ds(sid * RPT + z * CH, CH)], zsem).wait()
        pltpu.make_async_copy(zbuf.at[pl.ds(0, NZT)],
                              acc.at[pl.ds(sid * RPT + NZF * CH, NZT)],
                              zsem).wait()
        plsc.subcore_barrier()

        def round_body(jj, u):
            cur = (u % 2) * K
            nxt = K - cur
            ir_cur = u % 4
            ir_nxt = (u + 1) % 4
            ir_prev = (u + 3) % 4
            ir_load = (u + 2) % 4

            for k in range(K):
                pltpu.make_async_copy(
                    mloc.at[irb[ir_cur].at[0, k]], rows[cur + k],
                    gs[cur + k]).wait()
                pltpu.async_copy(rows[cur + k],
                                 acc.at[irb[ir_cur].at[1, k]],
                                 ss[cur + k], add=True)

            @pl.when(jj + 2 <= nrounds - 1)
            def _():
                pltpu.async_copy(idx_hbm.at[sid, rbase + jj + 2], irb[ir_load],
                                 isem[ir_load])

            @pl.when(jj >= 1)
            def _():
                for k in range(K):
                    pltpu.make_async_copy(
                        rows[nxt + k], acc.at[irb[ir_prev].at[1, k]],
                        ss[nxt + k]).wait()

            @pl.when(jj + 1 <= nrounds - 1)
            def _():
                pltpu.make_async_copy(idx_hbm.at[sid, rbase + jj + 1],
                                      irb[ir_nxt], isem[ir_nxt]).wait()
                for k in range(K):
                    pltpu.async_copy(mloc.at[irb[ir_nxt].at[0, k]],
                                     rows[nxt + k], gs[nxt + k])

        @pl.loop(0, nrounds, step=4)
        def _(j):
            for u in range(4):
                round_body(j + u, u)

        for k in range(K):
            pltpu.make_async_copy(
                rows[K + k], acc.at[irb[3].at[1, k]], ss[K + k]).wait()

        plsc.subcore_barrier()

        pltpu.sync_copy(
            acc.at[pl.ds(sid * RPT, RPT)],
            out_hbm.at[cid, pl.ds(sid * RPT, RPT)])

    return kernel_fn(m, idx5)


def _mm_body(h_ref, w_ref, o_ref):
    o_ref[...] = jnp.dot(h_ref[...], w_ref[...],
                         preferred_element_type=jnp.float32)


def _matmul(h, w):
    blk = 1000
    return pl.pallas_call(
        _mm_body,
        grid=(N // blk,),
        in_specs=[
            pl.BlockSpec((blk, D), lambda i: (i, 0)),
            pl.BlockSpec((D, D), lambda i: (0, 0)),
        ],
        out_specs=pl.BlockSpec((blk, D), lambda i: (i, 0)),
        out_shape=jax.ShapeDtypeStruct((N, D), jnp.float32),
    )(h, w)


def _gru_body(p_ref, h_ref, wih_ref, whh_ref, bih_ref, bhh_ref, o_ref):
    agg = p_ref[0] + p_ref[1]
    h = h_ref[...]
    gi = jnp.dot(agg, wih_ref[...],
                 preferred_element_type=jnp.float32) + bih_ref[...]
    gh = jnp.dot(h, whh_ref[...],
                 preferred_element_type=jnp.float32) + bhh_ref[...]
    r = jax.nn.sigmoid(gi[:, 0:D] + gh[:, 0:D])
    z = jax.nn.sigmoid(gi[:, D:2 * D] + gh[:, D:2 * D])
    n = jnp.tanh(gi[:, 2 * D:3 * D] + r * gh[:, 2 * D:3 * D])
    o_ref[...] = (1.0 - z) * n + z * h


def _gru(partials, h, w_ih_t, w_hh_t, b_ih2, b_hh2):
    blk = 1000
    return pl.pallas_call(
        _gru_body,
        grid=(N // blk,),
        in_specs=[
            pl.BlockSpec((NUM_SC, blk, D), lambda i: (0, i, 0)),
            pl.BlockSpec((blk, D), lambda i: (i, 0)),
            pl.BlockSpec((D, 3 * D), lambda i: (0, 0)),
            pl.BlockSpec((D, 3 * D), lambda i: (0, 0)),
            pl.BlockSpec((1, 3 * D), lambda i: (0, 0)),
            pl.BlockSpec((1, 3 * D), lambda i: (0, 0)),
        ],
        out_specs=pl.BlockSpec((blk, D), lambda i: (i, 0)),
        out_shape=jax.ShapeDtypeStruct((N, D), jnp.float32),
    )(partials, h, w_ih_t, w_hh_t, b_ih2, b_hh2)


def kernel(x, edge_index, weight, w_ih, w_hh, b_ih, b_hh):
    src = edge_index[0].astype(jnp.int32)
    dst = edge_index[1].astype(jnp.int32)
    pad = E_PAD - E
    src_p = jnp.concatenate([src, jnp.zeros((pad,), jnp.int32)])
    dst_p = jnp.concatenate([dst, jnp.full((pad,), N, jnp.int32)])
    idx5 = jnp.stack([src_p, dst_p]).reshape(2, NUM_TILES, RTOT, K, CH)
    idx5 = jnp.transpose(idx5, (1, 2, 0, 3, 4))

    w_ih_t = w_ih.T
    w_hh_t = w_hh.T
    b_ih2 = b_ih.reshape(1, 3 * D)
    b_hh2 = b_hh.reshape(1, 3 * D)

    h = x
    for i in range(L):
        m = _matmul(h, weight[i])
        partials = _sc_segment_sum(m, idx5)
        h = _gru(partials[:, :N, :], h, w_ih_t, w_hh_t, b_ih2, b_hh2)
    return h


# speedup vs baseline: 2.9612x; 2.9612x over previous
import functools

import jax
import jax.numpy as jnp
from jax import lax
from jax.experimental import pallas as pl
from jax.experimental.pallas import tpu as pltpu
from jax.experimental.pallas import tpu_sc as plsc

N = 10000
D = 128
E = 320000
L = 3

NUM_SC = 2
NUM_TILES = 16
NW = NUM_SC * NUM_TILES

CH = 32
K = 4
R0 = 80
R1 = 80
RTOT = R0 + R1
E_PAD = NUM_TILES * RTOT * K * CH
N_PAD = 10112
RPT = N_PAD // NUM_TILES
NZF = RPT // CH
NZT = RPT - NZF * CH


def _sc_segment_sum(m, idx5):
    mesh = plsc.VectorSubcoreMesh(core_axis_name="c", subcore_axis_name="s")

    @functools.partial(
        pl.kernel,
        out_type=jax.ShapeDtypeStruct((NUM_SC, N_PAD, D), jnp.float32),
        mesh=mesh,
        scratch_types=[
            [pltpu.VMEM((2, K, CH), jnp.int32) for _ in range(4)],
            [pltpu.VMEM((CH, D), jnp.float32) for _ in range(2 * K)],
            pltpu.VMEM_SHARED((N_PAD, D), jnp.float32),
            [pltpu.SemaphoreType.DMA for _ in range(2 * K)],
            [pltpu.SemaphoreType.DMA for _ in range(2 * K)],
            [pltpu.SemaphoreType.DMA for _ in range(4)],
            pltpu.SemaphoreType.DMA,
        ],
    )
    def kernel_fn(m_hbm, idx_hbm, out_hbm, irb, rows, acc, gs, ss, isem, zsem):
        cid = lax.axis_index("c")
        mloc = m_hbm
        sid = lax.axis_index("s")
        rbase = cid * R0
        nrounds = jnp.where(cid == 0, R0, R1)

        pltpu.async_copy(idx_hbm.at[sid, rbase + 0], irb[0], isem[0])
        pltpu.async_copy(idx_hbm.at[sid, rbase + 1], irb[1], isem[1])

        zbuf = rows[K]
        @pl.loop(0, CH)
        def _(r):
            @pl.loop(0, D, step=16)
            def _(c):
                zbuf[r, pl.ds(c, 16)] = jnp.zeros((16,), jnp.float32)

        for z in range(NZF):
            pltpu.async_copy(zbuf, acc.at[pl.ds(sid * RPT + z * CH, CH)], zsem)
        pltpu.async_copy(zbuf.at[pl.ds(0, NZT)],
                         acc.at[pl.ds(sid * RPT + NZF * CH, NZT)], zsem)

        pltpu.make_async_copy(idx_hbm.at[sid, rbase + 0], irb[0], isem[0]).wait()
        for k in range(K):
            pltpu.async_copy(mloc.at[irb[0].at[0, k]], rows[k], gs[k])

        for z in range(NZF):
            pltpu.make_async_copy(
                zbuf, acc.at[pl.ds(sid * RPT + z * CH, CH)], zsem).wait()
        pltpu.make_async_copy(zbuf.at[pl.ds(0, NZT)],
                              acc.at[pl.ds(sid * RPT + NZF * CH, NZT)],
                              zsem).wait()
        plsc.subcore_barrier()

        def round_body(jj, u):
            cur = (u % 2) * K
            nxt = K - cur
            ir_cur = u % 4
            ir_nxt = (u + 1) % 4
            ir_prev = (u + 3) % 4
            ir_load = (u + 2) % 4

            for k in range(K):
                pltpu.make_async_copy(
                    mloc.at[irb[ir_cur].at[0, k]], rows[cur + k],
                    gs[cur + k]).wait()
                pltpu.async_copy(rows[cur + k],
                                 acc.at[irb[ir_cur].at[1, k]],
                                 ss[cur + k], add=True)

            @pl.when(jj + 2 <= nrounds - 1)
            def _():
                pltpu.async_copy(idx_hbm.at[sid, rbase + jj + 2], irb[ir_load],
                                 isem[ir_load])

            @pl.when(jj >= 1)
            def _():
                for k in range(K):
                    pltpu.make_async_copy(
                        rows[nxt + k], acc.at[irb[ir_prev].at[1, k]],
                        ss[nxt + k]).wait()

            @pl.when(jj + 1 <= nrounds - 1)
            def _():
                pltpu.make_async_copy(idx_hbm.at[sid, rbase + jj + 1],
                                      irb[ir_nxt], isem[ir_nxt]).wait()
                for k in range(K):
                    pltpu.async_copy(mloc.at[irb[ir_nxt].at[0, k]],
                                     rows[nxt + k], gs[nxt + k])

        @pl.loop(0, nrounds, step=4)
        def _(j):
            for u in range(4):
                round_body(j + u, u)

        for k in range(K):
            pltpu.make_async_copy(
                rows[K + k], acc.at[irb[3].at[1, k]], ss[K + k]).wait()

        plsc.subcore_barrier()

        pltpu.sync_copy(
            acc.at[pl.ds(sid * RPT, RPT)],
            out_hbm.at[cid, pl.ds(sid * RPT, RPT)])

    return kernel_fn(m, idx5)


def _mm_body(h_ref, w_ref, o_ref):
    o_ref[...] = jnp.dot(h_ref[...], w_ref[...],
                         preferred_element_type=jnp.float32)


def _matmul(h, w):
    blk = 1000
    return pl.pallas_call(
        _mm_body,
        grid=(N // blk,),
        in_specs=[
            pl.BlockSpec((blk, D), lambda i: (i, 0)),
            pl.BlockSpec((D, D), lambda i: (0, 0)),
        ],
        out_specs=pl.BlockSpec((blk, D), lambda i: (i, 0)),
        out_shape=jax.ShapeDtypeStruct((N, D), jnp.float32),
    )(h, w)


def _gru_body(p_ref, h_ref, wih_ref, whh_ref, bih_ref, bhh_ref, o_ref):
    agg = p_ref[0] + p_ref[1]
    h = h_ref[...]
    gi = jnp.dot(agg, wih_ref[...],
                 preferred_element_type=jnp.float32) + bih_ref[...]
    gh = jnp.dot(h, whh_ref[...],
                 preferred_element_type=jnp.float32) + bhh_ref[...]
    r = jax.nn.sigmoid(gi[:, 0:D] + gh[:, 0:D])
    z = jax.nn.sigmoid(gi[:, D:2 * D] + gh[:, D:2 * D])
    n = jnp.tanh(gi[:, 2 * D:3 * D] + r * gh[:, 2 * D:3 * D])
    o_ref[...] = (1.0 - z) * n + z * h


def _gru(partials, h, w_ih_t, w_hh_t, b_ih2, b_hh2):
    blk = 1000
    return pl.pallas_call(
        _gru_body,
        grid=(N // blk,),
        in_specs=[
            pl.BlockSpec((NUM_SC, blk, D), lambda i: (0, i, 0)),
            pl.BlockSpec((blk, D), lambda i: (i, 0)),
            pl.BlockSpec((D, 3 * D), lambda i: (0, 0)),
            pl.BlockSpec((D, 3 * D), lambda i: (0, 0)),
            pl.BlockSpec((1, 3 * D), lambda i: (0, 0)),
            pl.BlockSpec((1, 3 * D), lambda i: (0, 0)),
        ],
        out_specs=pl.BlockSpec((blk, D), lambda i: (i, 0)),
        out_shape=jax.ShapeDtypeStruct((N, D), jnp.float32),
    )(partials, h, w_ih_t, w_hh_t, b_ih2, b_hh2)


def kernel(x, edge_index, weight, w_ih, w_hh, b_ih, b_hh):
    src = edge_index[0].astype(jnp.int32)
    dst = edge_index[1].astype(jnp.int32)
    pad = E_PAD - E
    pad_iota = jnp.arange(pad, dtype=jnp.int32)
    src_p = jnp.concatenate([src, pad_iota % N])
    dst_p = jnp.concatenate([dst, N + pad_iota % (N_PAD - N)])
    idx5 = jnp.stack([src_p, dst_p]).reshape(2, NUM_TILES, RTOT, K, CH)
    idx5 = jnp.transpose(idx5, (1, 2, 0, 3, 4))

    w_ih_t = w_ih.T
    w_hh_t = w_hh.T
    b_ih2 = b_ih.reshape(1, 3 * D)
    b_hh2 = b_hh.reshape(1, 3 * D)

    h = x
    for i in range(L):
        m = _matmul(h, weight[i])
        partials = _sc_segment_sum(m, idx5)
        h = _gru(partials[:, :N, :], h, w_ih_t, w_hh_t, b_ih2, b_hh2)
    return h


# speedup vs baseline: 3.0316x; 1.0238x over previous
import functools

import jax
import jax.numpy as jnp
from jax import lax
from jax.experimental import pallas as pl
from jax.experimental.pallas import tpu as pltpu
from jax.experimental.pallas import tpu_sc as plsc

N = 10000
D = 128
E = 320000
L = 3

NUM_SC = 2
NUM_TILES = 16
NW = NUM_SC * NUM_TILES

CH = 32
K = 4
R0 = 80
R1 = 80
RTOT = R0 + R1
E_PAD = NUM_TILES * RTOT * K * CH
N_PAD = 10112
RPT = N_PAD // NUM_TILES
NZF = RPT // CH
NZT = RPT - NZF * CH


def _sc_segment_sum(m, idx5):
    mesh = plsc.VectorSubcoreMesh(core_axis_name="c", subcore_axis_name="s")

    @functools.partial(
        pl.kernel,
        out_type=jax.ShapeDtypeStruct((NUM_SC, N_PAD, D), jnp.float32),
        mesh=mesh,
        scratch_types=[
            [pltpu.VMEM((2, K, CH), jnp.int32) for _ in range(4)],
            [pltpu.VMEM((CH, D), jnp.float32) for _ in range(2 * K)],
            pltpu.VMEM_SHARED((N_PAD, D), jnp.float32),
            [pltpu.SemaphoreType.DMA for _ in range(2 * K)],
            [pltpu.SemaphoreType.DMA for _ in range(2 * K)],
            [pltpu.SemaphoreType.DMA for _ in range(4)],
            pltpu.SemaphoreType.DMA,
        ],
    )
    def kernel_fn(m_hbm, idx_hbm, out_hbm, irb, rows, acc, gs, ss, isem, zsem):
        cid = lax.axis_index("c")
        mloc = m_hbm
        sid = lax.axis_index("s")
        rbase = cid * R0
        nrounds = jnp.where(cid == 0, R0, R1)

        pltpu.async_copy(idx_hbm.at[sid, rbase + 0], irb[0], isem[0])
        pltpu.async_copy(idx_hbm.at[sid, rbase + 1], irb[1], isem[1])

        zbuf = rows[K]
        @pl.loop(0, CH)
        def _(r):
            @pl.loop(0, D, step=16)
            def _(c):
                zbuf[r, pl.ds(c, 16)] = jnp.zeros((16,), jnp.float32)

        for z in range(NZF):
            pltpu.async_copy(zbuf, acc.at[pl.ds(sid * RPT + z * CH, CH)], zsem)
        pltpu.async_copy(zbuf.at[pl.ds(0, NZT)],
                         acc.at[pl.ds(sid * RPT + NZF * CH, NZT)], zsem)

        pltpu.make_async_copy(idx_hbm.at[sid, rbase + 0], irb[0], isem[0]).wait()
        for k in range(K):
            pltpu.async_copy(mloc.at[irb[0].at[0, k]], rows[k], gs[k])

        for z in range(NZF):
            pltpu.make_async_copy(
                zbuf, acc.at[pl.ds(sid * RPT + z * CH, CH)], zsem).wait()
        pltpu.make_async_copy(zbuf.at[pl.ds(0, NZT)],
                              acc.at[pl.ds(sid * RPT + NZF * CH, NZT)],
                              zsem).wait()
        plsc.subcore_barrier()

        def round_body(jj, u):
            cur = (u % 2) * K
            nxt = K - cur
            ir_cur = u % 4
            ir_nxt = (u + 1) % 4
            ir_prev = (u + 3) % 4
            ir_load = (u + 2) % 4

            for k in range(K):
                pltpu.make_async_copy(
                    mloc.at[irb[ir_cur].at[0, k]], rows[cur + k],
                    gs[cur + k]).wait()
                pltpu.async_copy(rows[cur + k],
                                 acc.at[irb[ir_cur].at[1, k]],
                                 ss[cur + k], add=True)

            @pl.when(jj + 2 <= nrounds - 1)
            def _():
                pltpu.async_copy(idx_hbm.at[sid, rbase + jj + 2], irb[ir_load],
                                 isem[ir_load])

            @pl.when(jj >= 1)
            def _():
                for k in range(K):
                    pltpu.make_async_copy(
                        rows[nxt + k], acc.at[irb[ir_prev].at[1, k]],
                        ss[nxt + k]).wait()

            @pl.when(jj + 1 <= nrounds - 1)
            def _():
                pltpu.make_async_copy(idx_hbm.at[sid, rbase + jj + 1],
                                      irb[ir_nxt], isem[ir_nxt]).wait()
                for k in range(K):
                    pltpu.async_copy(mloc.at[irb[ir_nxt].at[0, k]],
                                     rows[nxt + k], gs[nxt + k])

        @pl.loop(0, nrounds, step=4)
        def _(j):
            for u in range(4):
                round_body(j + u, u)

        for k in range(K):
            pltpu.make_async_copy(
                rows[K + k], acc.at[irb[3].at[1, k]], ss[K + k]).wait()

        plsc.subcore_barrier()

        pltpu.sync_copy(
            acc.at[pl.ds(sid * RPT, RPT)],
            out_hbm.at[cid, pl.ds(sid * RPT, RPT)])

    return kernel_fn(m, idx5)


def _mm_body(h_ref, w_ref, o_ref):
    o_ref[...] = jnp.dot(h_ref[...], w_ref[...],
                         preferred_element_type=jnp.float32)


def _matmul(h, w):
    blk = 1000
    return pl.pallas_call(
        _mm_body,
        grid=(N // blk,),
        in_specs=[
            pl.BlockSpec((blk, D), lambda i: (i, 0)),
            pl.BlockSpec((D, D), lambda i: (0, 0)),
        ],
        out_specs=pl.BlockSpec((blk, D), lambda i: (i, 0)),
        out_shape=jax.ShapeDtypeStruct((N, D), jnp.float32),
    )(h, w)


def _gh_body(h_ref, whh_ref, bhh_ref, o_ref):
    o_ref[...] = jnp.dot(h_ref[...], whh_ref[...],
                         preferred_element_type=jnp.float32) + bhh_ref[...]


def _gh(h, w_hh_t, b_hh2):
    blk = 1000
    return pl.pallas_call(
        _gh_body,
        grid=(N // blk,),
        in_specs=[
            pl.BlockSpec((blk, D), lambda i: (i, 0)),
            pl.BlockSpec((D, 3 * D), lambda i: (0, 0)),
            pl.BlockSpec((1, 3 * D), lambda i: (0, 0)),
        ],
        out_specs=pl.BlockSpec((blk, 3 * D), lambda i: (i, 0)),
        out_shape=jax.ShapeDtypeStruct((N, 3 * D), jnp.float32),
    )(h, w_hh_t, b_hh2)


def _gru_body(p_ref, h_ref, gh_ref, wih_ref, bih_ref, o_ref):
    agg = p_ref[0] + p_ref[1]
    h = h_ref[...]
    gh = gh_ref[...]
    gi = jnp.dot(agg, wih_ref[...],
                 preferred_element_type=jnp.float32) + bih_ref[...]
    r = jax.nn.sigmoid(gi[:, 0:D] + gh[:, 0:D])
    z = jax.nn.sigmoid(gi[:, D:2 * D] + gh[:, D:2 * D])
    n = jnp.tanh(gi[:, 2 * D:3 * D] + r * gh[:, 2 * D:3 * D])
    o_ref[...] = (1.0 - z) * n + z * h


def _gru(partials, h, gh, w_ih_t, b_ih2):
    blk = 1000
    return pl.pallas_call(
        _gru_body,
        grid=(N // blk,),
        in_specs=[
            pl.BlockSpec((NUM_SC, blk, D), lambda i: (0, i, 0)),
            pl.BlockSpec((blk, D), lambda i: (i, 0)),
            pl.BlockSpec((blk, 3 * D), lambda i: (i, 0)),
            pl.BlockSpec((D, 3 * D), lambda i: (0, 0)),
            pl.BlockSpec((1, 3 * D), lambda i: (0, 0)),
        ],
        out_specs=pl.BlockSpec((blk, D), lambda i: (i, 0)),
        out_shape=jax.ShapeDtypeStruct((N, D), jnp.float32),
    )(partials, h, gh, w_ih_t, b_ih2)


def kernel(x, edge_index, weight, w_ih, w_hh, b_ih, b_hh):
    src = edge_index[0].astype(jnp.int32)
    dst = edge_index[1].astype(jnp.int32)
    pad = E_PAD - E
    pad_iota = jnp.arange(pad, dtype=jnp.int32)
    src_p = jnp.concatenate([src, pad_iota % N])
    dst_p = jnp.concatenate([dst, N + pad_iota % (N_PAD - N)])
    idx5 = jnp.stack([src_p, dst_p]).reshape(2, NUM_TILES, RTOT, K, CH)
    idx5 = jnp.transpose(idx5, (1, 2, 0, 3, 4))

    w_ih_t = w_ih.T
    w_hh_t = w_hh.T
    b_ih2 = b_ih.reshape(1, 3 * D)
    b_hh2 = b_hh.reshape(1, 3 * D)

    h = x
    for i in range(L):
        m = _matmul(h, weight[i])
        gh = _gh(h, w_hh_t, b_hh2)
        partials = _sc_segment_sum(m, idx5)
        h = _gru(partials, h, gh, w_ih_t, b_ih2)
    return h


# speedup vs baseline: 3.1499x; 1.0390x over previous
import functools

import jax
import jax.numpy as jnp
from jax import lax
from jax.experimental import pallas as pl
from jax.experimental.pallas import tpu as pltpu
from jax.experimental.pallas import tpu_sc as plsc

N = 10000
D = 128
E = 320000
L = 3

NUM_SC = 2
NUM_TILES = 16
NW = NUM_SC * NUM_TILES

CH = 64
K = 2
R0 = 80
R1 = 80
RTOT = R0 + R1
E_PAD = NUM_TILES * RTOT * K * CH
N_PAD = 10112
RPT = N_PAD // NUM_TILES
NZF = RPT // CH
NZT = RPT - NZF * CH


def _sc_segment_sum(m, idx5):
    mesh = plsc.VectorSubcoreMesh(core_axis_name="c", subcore_axis_name="s")

    @functools.partial(
        pl.kernel,
        out_type=jax.ShapeDtypeStruct((NUM_SC, N_PAD, D), jnp.float32),
        mesh=mesh,
        scratch_types=[
            [pltpu.VMEM((2, K, CH), jnp.int32) for _ in range(4)],
            [pltpu.VMEM((CH, D), jnp.float32) for _ in range(2 * K)],
            pltpu.VMEM_SHARED((N_PAD, D), jnp.float32),
            [pltpu.SemaphoreType.DMA for _ in range(2 * K)],
            [pltpu.SemaphoreType.DMA for _ in range(2 * K)],
            [pltpu.SemaphoreType.DMA for _ in range(4)],
            pltpu.SemaphoreType.DMA,
        ],
    )
    def kernel_fn(m_hbm, idx_hbm, out_hbm, irb, rows, acc, gs, ss, isem, zsem):
        cid = lax.axis_index("c")
        mloc = m_hbm
        sid = lax.axis_index("s")
        rbase = cid * R0
        nrounds = jnp.where(cid == 0, R0, R1)

        pltpu.async_copy(idx_hbm.at[sid, rbase + 0], irb[0], isem[0])
        pltpu.async_copy(idx_hbm.at[sid, rbase + 1], irb[1], isem[1])

        zbuf = rows[K]
        @pl.loop(0, CH)
        def _(r):
            @pl.loop(0, D, step=16)
            def _(c):
                zbuf[r, pl.ds(c, 16)] = jnp.zeros((16,), jnp.float32)

        for z in range(NZF):
            pltpu.async_copy(zbuf, acc.at[pl.ds(sid * RPT + z * CH, CH)], zsem)
        pltpu.async_copy(zbuf.at[pl.ds(0, NZT)],
                         acc.at[pl.ds(sid * RPT + NZF * CH, NZT)], zsem)

        pltpu.make_async_copy(idx_hbm.at[sid, rbase + 0], irb[0], isem[0]).wait()
        for k in range(K):
            pltpu.async_copy(mloc.at[irb[0].at[0, k]], rows[k], gs[k])

        for z in range(NZF):
            pltpu.make_async_copy(
                zbuf, acc.at[pl.ds(sid * RPT + z * CH, CH)], zsem).wait()
        pltpu.make_async_copy(zbuf.at[pl.ds(0, NZT)],
                              acc.at[pl.ds(sid * RPT + NZF * CH, NZT)],
                              zsem).wait()
        plsc.subcore_barrier()

        def round_body(jj, u):
            cur = (u % 2) * K
            nxt = K - cur
            ir_cur = u % 4
            ir_nxt = (u + 1) % 4
            ir_prev = (u + 3) % 4
            ir_load = (u + 2) % 4

            for k in range(K):
                pltpu.make_async_copy(
                    mloc.at[irb[ir_cur].at[0, k]], rows[cur + k],
                    gs[cur + k]).wait()
                pltpu.async_copy(rows[cur + k],
                                 acc.at[irb[ir_cur].at[1, k]],
                                 ss[cur + k], add=True)

            @pl.when(jj + 2 <= nrounds - 1)
            def _():
                pltpu.async_copy(idx_hbm.at[sid, rbase + jj + 2], irb[ir_load],
                                 isem[ir_load])

            @pl.when(jj >= 1)
            def _():
                for k in range(K):
                    pltpu.make_async_copy(
                        rows[nxt + k], acc.at[irb[ir_prev].at[1, k]],
                        ss[nxt + k]).wait()

            @pl.when(jj + 1 <= nrounds - 1)
            def _():
                pltpu.make_async_copy(idx_hbm.at[sid, rbase + jj + 1],
                                      irb[ir_nxt], isem[ir_nxt]).wait()
                for k in range(K):
                    pltpu.async_copy(mloc.at[irb[ir_nxt].at[0, k]],
                                     rows[nxt + k], gs[nxt + k])

        @pl.loop(0, nrounds, step=4)
        def _(j):
            for u in range(4):
                round_body(j + u, u)

        for k in range(K):
            pltpu.make_async_copy(
                rows[K + k], acc.at[irb[3].at[1, k]], ss[K + k]).wait()

        plsc.subcore_barrier()

        pltpu.sync_copy(
            acc.at[pl.ds(sid * RPT, RPT)],
            out_hbm.at[cid, pl.ds(sid * RPT, RPT)])

    return kernel_fn(m, idx5)


def _mm_body(h_ref, w_ref, o_ref):
    o_ref[...] = jnp.dot(h_ref[...], w_ref[...],
                         preferred_element_type=jnp.float32)


def _matmul(h, w):
    blk = 1000
    return pl.pallas_call(
        _mm_body,
        grid=(N // blk,),
        in_specs=[
            pl.BlockSpec((blk, D), lambda i: (i, 0)),
            pl.BlockSpec((D, D), lambda i: (0, 0)),
        ],
        out_specs=pl.BlockSpec((blk, D), lambda i: (i, 0)),
        out_shape=jax.ShapeDtypeStruct((N, D), jnp.float32),
    )(h, w)


def _gh_body(h_ref, whh_ref, bhh_ref, o_ref):
    o_ref[...] = jnp.dot(h_ref[...], whh_ref[...],
                         preferred_element_type=jnp.float32) + bhh_ref[...]


def _gh(h, w_hh_t, b_hh2):
    blk = 1000
    return pl.pallas_call(
        _gh_body,
        grid=(N // blk,),
        in_specs=[
            pl.BlockSpec((blk, D), lambda i: (i, 0)),
            pl.BlockSpec((D, 3 * D), lambda i: (0, 0)),
            pl.BlockSpec((1, 3 * D), lambda i: (0, 0)),
        ],
        out_specs=pl.BlockSpec((blk, 3 * D), lambda i: (i, 0)),
        out_shape=jax.ShapeDtypeStruct((N, 3 * D), jnp.float32),
    )(h, w_hh_t, b_hh2)


def _gru_body(p_ref, h_ref, gh_ref, wih_ref, bih_ref, o_ref):
    agg = p_ref[0] + p_ref[1]
    h = h_ref[...]
    gh = gh_ref[...]
    gi = jnp.dot(agg, wih_ref[...],
                 preferred_element_type=jnp.float32) + bih_ref[...]
    r = jax.nn.sigmoid(gi[:, 0:D] + gh[:, 0:D])
    z = jax.nn.sigmoid(gi[:, D:2 * D] + gh[:, D:2 * D])
    n = jnp.tanh(gi[:, 2 * D:3 * D] + r * gh[:, 2 * D:3 * D])
    o_ref[...] = (1.0 - z) * n + z * h


def _gru(partials, h, gh, w_ih_t, b_ih2):
    blk = 1000
    return pl.pallas_call(
        _gru_body,
        grid=(N // blk,),
        in_specs=[
            pl.BlockSpec((NUM_SC, blk, D), lambda i: (0, i, 0)),
            pl.BlockSpec((blk, D), lambda i: (i, 0)),
            pl.BlockSpec((blk, 3 * D), lambda i: (i, 0)),
            pl.BlockSpec((D, 3 * D), lambda i: (0, 0)),
            pl.BlockSpec((1, 3 * D), lambda i: (0, 0)),
        ],
        out_specs=pl.BlockSpec((blk, D), lambda i: (i, 0)),
        out_shape=jax.ShapeDtypeStruct((N, D), jnp.float32),
    )(partials, h, gh, w_ih_t, b_ih2)


def kernel(x, edge_index, weight, w_ih, w_hh, b_ih, b_hh):
    src = edge_index[0].astype(jnp.int32)
    dst = edge_index[1].astype(jnp.int32)
    pad = E_PAD - E
    pad_iota = jnp.arange(pad, dtype=jnp.int32)
    src_p = jnp.concatenate([src, pad_iota % N])
    dst_p = jnp.concatenate([dst, N + pad_iota % (N_PAD - N)])
    idx5 = jnp.stack([src_p, dst_p]).reshape(2, NUM_TILES, RTOT, K, CH)
    idx5 = jnp.transpose(idx5, (1, 2, 0, 3, 4))

    w_ih_t = w_ih.T
    w_hh_t = w_hh.T
    b_ih2 = b_ih.reshape(1, 3 * D)
    b_hh2 = b_hh.reshape(1, 3 * D)

    h = x
    for i in range(L):
        m = _matmul(h, weight[i])
        gh = _gh(h, w_hh_t, b_hh2)
        partials = _sc_segment_sum(m, idx5)
        h = _gru(partials, h, gh, w_ih_t, b_ih2)
    return h


# speedup vs baseline: 3.2745x; 1.0396x over previous
import functools

import jax
import jax.numpy as jnp
from jax import lax
from jax.experimental import pallas as pl
from jax.experimental.pallas import tpu as pltpu
from jax.experimental.pallas import tpu_sc as plsc

N = 10000
D = 128
E = 320000
L = 3

NUM_SC = 2
NUM_TILES = 16
NW = NUM_SC * NUM_TILES

CH = 80
K = 2
R0 = 64
R1 = 64
RTOT = R0 + R1
E_PAD = NUM_TILES * RTOT * K * CH
N_PAD = 10112
RPT = N_PAD // NUM_TILES
NZF = RPT // CH
NZT = RPT - NZF * CH


def _sc_segment_sum(m, idx5):
    mesh = plsc.VectorSubcoreMesh(core_axis_name="c", subcore_axis_name="s")

    @functools.partial(
        pl.kernel,
        out_type=jax.ShapeDtypeStruct((NUM_SC, N_PAD, D), jnp.float32),
        mesh=mesh,
        scratch_types=[
            [pltpu.VMEM((2, K, CH), jnp.int32) for _ in range(4)],
            [pltpu.VMEM((CH, D), jnp.float32) for _ in range(2 * K)],
            pltpu.VMEM_SHARED((N_PAD, D), jnp.float32),
            [pltpu.SemaphoreType.DMA for _ in range(2 * K)],
            [pltpu.SemaphoreType.DMA for _ in range(2 * K)],
            [pltpu.SemaphoreType.DMA for _ in range(4)],
            pltpu.SemaphoreType.DMA,
        ],
    )
    def kernel_fn(m_hbm, idx_hbm, out_hbm, irb, rows, acc, gs, ss, isem, zsem):
        cid = lax.axis_index("c")
        mloc = m_hbm
        sid = lax.axis_index("s")
        rbase = cid * R0
        nrounds = jnp.where(cid == 0, R0, R1)

        pltpu.async_copy(idx_hbm.at[sid, rbase + 0], irb[0], isem[0])
        pltpu.async_copy(idx_hbm.at[sid, rbase + 1], irb[1], isem[1])

        zbuf = rows[K]
        @pl.loop(0, CH)
        def _(r):
            @pl.loop(0, D, step=16)
            def _(c):
                zbuf[r, pl.ds(c, 16)] = jnp.zeros((16,), jnp.float32)

        for z in range(NZF):
            pltpu.async_copy(zbuf, acc.at[pl.ds(sid * RPT + z * CH, CH)], zsem)
        pltpu.async_copy(zbuf.at[pl.ds(0, NZT)],
                         acc.at[pl.ds(sid * RPT + NZF * CH, NZT)], zsem)

        pltpu.make_async_copy(idx_hbm.at[sid, rbase + 0], irb[0], isem[0]).wait()
        for k in range(K):
            pltpu.async_copy(mloc.at[irb[0].at[0, k]], rows[k], gs[k])

        for z in range(NZF):
            pltpu.make_async_copy(
                zbuf, acc.at[pl.ds(sid * RPT + z * CH, CH)], zsem).wait()
        pltpu.make_async_copy(zbuf.at[pl.ds(0, NZT)],
                              acc.at[pl.ds(sid * RPT + NZF * CH, NZT)],
                              zsem).wait()
        plsc.subcore_barrier()

        def round_body(jj, u):
            cur = (u % 2) * K
            nxt = K - cur
            ir_cur = u % 4
            ir_nxt = (u + 1) % 4
            ir_prev = (u + 3) % 4
            ir_load = (u + 2) % 4

            for k in range(K):
                pltpu.make_async_copy(
                    mloc.at[irb[ir_cur].at[0, k]], rows[cur + k],
                    gs[cur + k]).wait()
                pltpu.async_copy(rows[cur + k],
                                 acc.at[irb[ir_cur].at[1, k]],
                                 ss[cur + k], add=True)

            @pl.when(jj + 2 <= nrounds - 1)
            def _():
                pltpu.async_copy(idx_hbm.at[sid, rbase + jj + 2], irb[ir_load],
                                 isem[ir_load])

            @pl.when(jj >= 1)
            def _():
                for k in range(K):
                    pltpu.make_async_copy(
                        rows[nxt + k], acc.at[irb[ir_prev].at[1, k]],
                        ss[nxt + k]).wait()

            @pl.when(jj + 1 <= nrounds - 1)
            def _():
                pltpu.make_async_copy(idx_hbm.at[sid, rbase + jj + 1],
                                      irb[ir_nxt], isem[ir_nxt]).wait()
                for k in range(K):
                    pltpu.async_copy(mloc.at[irb[ir_nxt].at[0, k]],
                                     rows[nxt + k], gs[nxt + k])

        @pl.loop(0, nrounds, step=4)
        def _(j):
            for u in range(4):
                round_body(j + u, u)

        for k in range(K):
            pltpu.make_async_copy(
                rows[K + k], acc.at[irb[3].at[1, k]], ss[K + k]).wait()

        plsc.subcore_barrier()

        pltpu.sync_copy(
            acc.at[pl.ds(sid * RPT, RPT)],
            out_hbm.at[cid, pl.ds(sid * RPT, RPT)])

    return kernel_fn(m, idx5)


def _mm_body(h_ref, w_ref, o_ref):
    o_ref[...] = jnp.dot(h_ref[...], w_ref[...],
                         preferred_element_type=jnp.float32)


def _matmul(h, w):
    blk = 1000
    return pl.pallas_call(
        _mm_body,
        grid=(N // blk,),
        in_specs=[
            pl.BlockSpec((blk, D), lambda i: (i, 0)),
            pl.BlockSpec((D, D), lambda i: (0, 0)),
        ],
        out_specs=pl.BlockSpec((blk, D), lambda i: (i, 0)),
        out_shape=jax.ShapeDtypeStruct((N, D), jnp.float32),
    )(h, w)


def _gh_body(h_ref, whh_ref, bhh_ref, o_ref):
    o_ref[...] = jnp.dot(h_ref[...], whh_ref[...],
                         preferred_element_type=jnp.float32) + bhh_ref[...]


def _gh(h, w_hh_t, b_hh2):
    blk = 1000
    return pl.pallas_call(
        _gh_body,
        grid=(N // blk,),
        in_specs=[
            pl.BlockSpec((blk, D), lambda i: (i, 0)),
            pl.BlockSpec((D, 3 * D), lambda i: (0, 0)),
            pl.BlockSpec((1, 3 * D), lambda i: (0, 0)),
        ],
        out_specs=pl.BlockSpec((blk, 3 * D), lambda i: (i, 0)),
        out_shape=jax.ShapeDtypeStruct((N, 3 * D), jnp.float32),
    )(h, w_hh_t, b_hh2)


def _gru_body(p_ref, h_ref, gh_ref, wih_ref, bih_ref, o_ref):
    agg = p_ref[0] + p_ref[1]
    h = h_ref[...]
    gh = gh_ref[...]
    gi = jnp.dot(agg, wih_ref[...],
                 preferred_element_type=jnp.float32) + bih_ref[...]
    r = jax.nn.sigmoid(gi[:, 0:D] + gh[:, 0:D])
    z = jax.nn.sigmoid(gi[:, D:2 * D] + gh[:, D:2 * D])
    n = jnp.tanh(gi[:, 2 * D:3 * D] + r * gh[:, 2 * D:3 * D])
    o_ref[...] = (1.0 - z) * n + z * h


def _gru(partials, h, gh, w_ih_t, b_ih2):
    blk = 1000
    return pl.pallas_call(
        _gru_body,
        grid=(N // blk,),
        in_specs=[
            pl.BlockSpec((NUM_SC, blk, D), lambda i: (0, i, 0)),
            pl.BlockSpec((blk, D), lambda i: (i, 0)),
            pl.BlockSpec((blk, 3 * D), lambda i: (i, 0)),
            pl.BlockSpec((D, 3 * D), lambda i: (0, 0)),
            pl.BlockSpec((1, 3 * D), lambda i: (0, 0)),
        ],
        out_specs=pl.BlockSpec((blk, D), lambda i: (i, 0)),
        out_shape=jax.ShapeDtypeStruct((N, D), jnp.float32),
    )(partials, h, gh, w_ih_t, b_ih2)


def kernel(x, edge_index, weight, w_ih, w_hh, b_ih, b_hh):
    src = edge_index[0].astype(jnp.int32)
    dst = edge_index[1].astype(jnp.int32)
    pad = E_PAD - E
    pad_iota = jnp.arange(pad, dtype=jnp.int32)
    src_p = jnp.concatenate([src, pad_iota % N])
    dst_p = jnp.concatenate([dst, N + pad_iota % (N_PAD - N)])
    idx5 = jnp.stack([src_p, dst_p]).reshape(2, NUM_TILES, RTOT, K, CH)
    idx5 = jnp.transpose(idx5, (1, 2, 0, 3, 4))

    w_ih_t = w_ih.T
    w_hh_t = w_hh.T
    b_ih2 = b_ih.reshape(1, 3 * D)
    b_hh2 = b_hh.reshape(1, 3 * D)

    h = x
    for i in range(L):
        m = _matmul(h, weight[i])
        gh = _gh(h, w_hh_t, b_hh2)
        partials = _sc_segment_sum(m, idx5)
        h = _gru(partials, h, gh, w_ih_t, b_ih2)
    return h
